# baseline (speedup 1.0000x reference)
import functools

import jax
import jax.numpy as jnp
from jax import lax
from jax.experimental import pallas as pl
from jax.experimental.pallas import tpu as pltpu
from jax.experimental.pallas import tpu_sc as plsc

LATENT = 128
BATCH = 16384
SC_ROWS = 8192
CHUNKS = 2
TC_BLOCK = 2048


def _build_sc():
    info = plsc.get_sparse_core_info()
    nc, ns = info.num_cores, info.num_subcores
    nw = nc * ns
    bpw = SC_ROWS // nw
    nk = LATENT // 16
    rows_per_chunk = bpw // CHUNKS
    groups_per_chunk = rows_per_chunk // 16
    mesh = plsc.VectorSubcoreMesh(core_axis_name="c", subcore_axis_name="s")

    @functools.partial(
        pl.kernel,
        mesh=mesh,
        out_type=jax.ShapeDtypeStruct((BATCH, LATENT), jnp.float32),
        scratch_types=[
            pltpu.VMEM((bpw,), jnp.int32),
            pltpu.VMEM((2, LATENT), jnp.float32),
            pltpu.VMEM((bpw, LATENT), jnp.float32),
            pltpu.SemaphoreType.DMA,
            pltpu.SemaphoreType.DMA,
        ],
    )
    def emb(idx_hbm, table_hbm, out_hbm, idx_v, tab_v, rows_v, sem, psem):
        wid = lax.axis_index("c") * ns + lax.axis_index("s")
        base = wid * bpw
        idx_cp = pltpu.async_copy(idx_hbm.at[pl.ds(base, bpw)], idx_v, psem)
        pltpu.sync_copy(table_hbm, tab_v)
        idx_cp.wait()
        t0 = [tab_v[0, pl.ds(16 * k, 16)] for k in range(nk)]
        d1 = [tab_v[1, pl.ds(16 * k, 16)] - t0[k] for k in range(nk)]

        def body(g, carry):
            r0 = g * 16
            idxf = idx_v[pl.ds(r0, 16)].astype(jnp.float32)
            for j in range(16):
                bc = jnp.full((16,), idxf[j], dtype=jnp.float32)
                for k in range(nk):
                    rows_v[r0 + j, pl.ds(16 * k, 16)] = t0[k] + bc * d1[k]
            return carry

        copies = []
        for c in range(CHUNKS):
            lax.fori_loop(
                c * groups_per_chunk, (c + 1) * groups_per_chunk, body, 0
            )
            copies.append(
                pltpu.async_copy(
                    rows_v.at[pl.ds(c * rows_per_chunk, rows_per_chunk)],
                    out_hbm.at[pl.ds(base + c * rows_per_chunk, rows_per_chunk)],
                    sem,
                )
            )
        for cp in copies:
            cp.wait()

    return emb


_emb_sc = _build_sc()


def _tc_body(x_ref, tab_ref, _, o_ref):
    mask = x_ref[...] != 0
    o_ref[...] = jnp.where(mask, tab_ref[1:2, :], tab_ref[0:1, :])


def _tc_fill(x2d, table, scbuf):
    nb = (BATCH - SC_ROWS) // TC_BLOCK
    hb = SC_ROWS // TC_BLOCK
    return pl.pallas_call(
        _tc_body,
        grid=(nb,),
        in_specs=[
            pl.BlockSpec((TC_BLOCK, 1), lambda i: (hb + i, 0)),
            pl.BlockSpec((2, LATENT), lambda i: (0, 0)),
            pl.BlockSpec(memory_space=pl.ANY),
        ],
        out_specs=pl.BlockSpec((TC_BLOCK, LATENT), lambda i: (hb + i, 0)),
        out_shape=jax.ShapeDtypeStruct((BATCH, LATENT), jnp.float32),
        input_output_aliases={2: 0},
    )(x2d, table, scbuf)


@jax.jit
def kernel(x, table):
    x32 = x.astype(jnp.int32)
    scbuf = _emb_sc(x32, table)
    out = _tc_fill(x32.reshape(-1, 1), table, scbuf)
    return out.reshape(-1, LATENT, 1, 1)


# speedup vs baseline: 1.1951x; 1.1951x over previous
import functools

import jax
import jax.numpy as jnp
from jax import lax
from jax.experimental import pallas as pl
from jax.experimental.pallas import tpu as pltpu
from jax.experimental.pallas import tpu_sc as plsc

LATENT = 128
BATCH = 16384
CHUNKS = 2


def _build():
    info = plsc.get_sparse_core_info()
    nc, ns = info.num_cores, info.num_subcores
    nw = nc * ns
    bpw = BATCH // nw
    nk = LATENT // 16
    rows_per_chunk = bpw // CHUNKS
    groups_per_chunk = rows_per_chunk // 16
    mesh = plsc.VectorSubcoreMesh(core_axis_name="c", subcore_axis_name="s")

    @functools.partial(
        pl.kernel,
        mesh=mesh,
        out_type=jax.ShapeDtypeStruct((BATCH, LATENT), jnp.float32),
        scratch_types=[
            pltpu.VMEM((bpw,), jnp.int32),
            pltpu.VMEM((2, LATENT), jnp.float32),
            pltpu.VMEM((bpw, LATENT), jnp.float32),
            pltpu.SemaphoreType.DMA,
            pltpu.SemaphoreType.DMA,
        ],
    )
    def emb(idx_hbm, table_hbm, out_hbm, idx_v, tab_v, rows_v, sem, psem):
        wid = lax.axis_index("c") * ns + lax.axis_index("s")
        base = wid * bpw
        idx_cp = pltpu.async_copy(idx_hbm.at[pl.ds(base, bpw)], idx_v, psem)
        pltpu.sync_copy(table_hbm, tab_v)
        idx_cp.wait()
        t0 = [tab_v[0, pl.ds(16 * k, 16)] for k in range(nk)]
        d1 = [tab_v[1, pl.ds(16 * k, 16)] - t0[k] for k in range(nk)]

        def body(g, carry):
            r0 = g * 16
            idxf = idx_v[pl.ds(r0, 16)].astype(jnp.float32)
            for j in range(16):
                bc = jnp.full((16,), idxf[j], dtype=jnp.float32)
                for k in range(nk):
                    rows_v[r0 + j, pl.ds(16 * k, 16)] = t0[k] + bc * d1[k]
            return carry

        copies = []
        for c in range(CHUNKS):
            lax.fori_loop(
                c * groups_per_chunk, (c + 1) * groups_per_chunk, body, 0,
                unroll=2,
            )
            copies.append(
                pltpu.async_copy(
                    rows_v.at[pl.ds(c * rows_per_chunk, rows_per_chunk)],
                    out_hbm.at[pl.ds(base + c * rows_per_chunk, rows_per_chunk)],
                    sem,
                )
            )
        for cp in copies:
            cp.wait()

    return emb


_emb = _build()


@jax.jit
def kernel(x, table):
    out = _emb(x.astype(jnp.int32), table)
    return out.reshape(-1, LATENT, 1, 1)
